# baseline (speedup 1.0000x reference)
import functools

import jax
import jax.numpy as jnp
from jax import lax
from jax.experimental import pallas as pl
from jax.experimental.pallas import tpu as pltpu
from jax.experimental.pallas import tpu_sc as plsc

B = 16384
ID_V = 100001
EMB = 32
SEQ = 20
NW = 32
RPW = B // NW
NG = RPW // 16

_COMPILER_PARAMS = pltpu.CompilerParams(
    use_tc_tiling_on_sc=False, needs_layout_passes=False)


def _wid_base():
  c = lax.axis_index("c")
  s = lax.axis_index("s")
  return (s * 2 + c) * RPW


def _text_body(tokT_hbm, txttab_hbm, txtT_hbm,
               toks_v, acc_v, txtT_v, a_v, b_v, row0_v, sem_tx):
  base = _wid_base()

  def _zero(r, _):
    z = jnp.zeros((16,), jnp.float32)
    acc_v[r, pl.ds(0, 16)] = z
    acc_v[r, pl.ds(16, 16)] = z
    return _
  lax.fori_loop(0, RPW, _zero, None)

  pltpu.sync_copy(tokT_hbm.at[:, pl.ds(base, RPW)], toks_v)
  pltpu.sync_copy(txttab_hbm.at[pl.ds(0, 1)], row0_v)

  tx_copies = []
  for j in range(SEQ):
    cp = pltpu.make_async_copy(txttab_hbm.at[toks_v.at[j]], acc_v, sem_tx)
    cp.start(add=True)
    tx_copies.append(cp)

  one = jnp.ones((16,), jnp.float32)
  zero = jnp.zeros((16,), jnp.float32)

  def _count(g, _):
    r16 = pl.multiple_of(g * 16, 16)
    cnt = jnp.zeros((16,), jnp.float32)
    for j in range(SEQ):
      v = toks_v[j, pl.ds(r16, 16)]
      cnt = cnt + jnp.where(v != 0, one, zero)
    denom = jnp.maximum(cnt, 1.0)
    a_v[pl.ds(r16, 16)] = 1.0 / denom
    b_v[pl.ds(r16, 16)] = (float(SEQ) - cnt) / denom
    return _
  lax.fori_loop(0, NG, _count, None)

  for cp in tx_copies:
    cp.wait()

  r0_lo = row0_v[0, pl.ds(0, 16)]
  r0_hi = row0_v[0, pl.ds(16, 16)]
  lanes = lax.iota(jnp.int32, 16)

  def _scale(g, _):
    r16 = pl.multiple_of(g * 16, 16)
    rows = r16 + lanes
    avec = a_v[pl.ds(r16, 16)]
    bvec = b_v[pl.ds(r16, 16)]
    for cc in range(EMB):
      col = jnp.full((16,), cc, jnp.int32)
      accv = plsc.load_gather(acc_v, [rows, col])
      r0c = r0_lo[cc] if cc < 16 else r0_hi[cc - 16]
      txtT_v[cc, pl.ds(r16, 16)] = accv * avec - r0c * bvec
    return _
  lax.fori_loop(0, NG, _scale, None)

  pltpu.sync_copy(txtT_v, txtT_hbm.at[:, pl.ds(base, RPW)])


def _id_body(idx_hbm, idtabT_hbm, idT_hbm,
             ididx_v, gidx_v, idcols_v, sem_id):
  base = _wid_base()
  pltpu.sync_copy(idx_hbm.at[pl.ds(base, RPW)], ididx_v)

  def _gidx(g, _):
    r16 = pl.multiple_of(g * 16, 16)
    iv = ididx_v[pl.ds(r16, 16)]
    for cc in range(EMB):
      gidx_v[cc, pl.ds(r16, 16)] = iv + (cc * ID_V)
    return _
  lax.fori_loop(0, NG, _gidx, None)

  id_copies = []
  for cc in range(EMB):
    cp = pltpu.make_async_copy(
        idtabT_hbm.at[gidx_v.at[cc]], idcols_v.at[cc], sem_id)
    cp.start()
    id_copies.append(cp)
  for cp in id_copies:
    cp.wait()

  pltpu.sync_copy(idcols_v, idT_hbm.at[:, pl.ds(base, RPW)])


@functools.partial(jax.jit, static_argnums=())
def _run(jnrl_id_idx, tokT, idtabT_flat, text_table):
  mesh = plsc.VectorSubcoreMesh(core_axis_name="c", subcore_axis_name="s")
  k_text = pl.kernel(
      _text_body,
      out_type=jax.ShapeDtypeStruct((EMB, B), jnp.float32),
      mesh=mesh,
      compiler_params=_COMPILER_PARAMS,
      scratch_types=[
          pltpu.VMEM((SEQ, RPW), jnp.int32),
          pltpu.VMEM((RPW, EMB), jnp.float32),
          pltpu.VMEM((EMB, RPW), jnp.float32),
          pltpu.VMEM((RPW,), jnp.float32),
          pltpu.VMEM((RPW,), jnp.float32),
          pltpu.VMEM((1, EMB), jnp.float32),
          pltpu.SemaphoreType.DMA,
      ],
  )
  k_id = pl.kernel(
      _id_body,
      out_type=jax.ShapeDtypeStruct((EMB, B), jnp.float32),
      mesh=mesh,
      compiler_params=_COMPILER_PARAMS,
      scratch_types=[
          pltpu.VMEM((RPW,), jnp.int32),
          pltpu.VMEM((EMB, RPW), jnp.int32),
          pltpu.VMEM((EMB, RPW), jnp.float32),
          pltpu.SemaphoreType.DMA,
      ],
  )
  txtT = k_text(tokT, text_table)
  idT = k_id(jnrl_id_idx, idtabT_flat)
  return jnp.transpose(jnp.concatenate([idT, txtT], axis=0))


def kernel(jnrl_id_idx, text_token_ids, id_table, text_table):
  tokT = jnp.transpose(text_token_ids)
  idtabT_flat = jnp.transpose(id_table).reshape(-1)
  return _run(jnrl_id_idx, tokT, idtabT_flat, text_table)


# speedup vs baseline: 1.1185x; 1.1185x over previous
import functools

import jax
import jax.numpy as jnp
from jax import lax
from jax.experimental import pallas as pl
from jax.experimental.pallas import tpu as pltpu
from jax.experimental.pallas import tpu_sc as plsc

B = 16384
ID_V = 100001
EMB = 32
SEQ = 20
NW = 32
RPW = B // NW
NG = RPW // 16


def _sc_body(idx_hbm, tokT_hbm, idtabT_hbm, txttab_hbm, outT_hbm,
             toks_v, ididx_v, gidx_v, idcols_v, acc_v, txtT_v, a_v, b_v,
             row0_v, sem_st, sem_id, sem_tx):
  c = lax.axis_index("c")
  s = lax.axis_index("s")
  base = (s * 2 + c) * RPW

  st_copies = [
      pltpu.make_async_copy(tokT_hbm.at[:, pl.ds(base, RPW)], toks_v,
                            sem_st),
      pltpu.make_async_copy(idx_hbm.at[pl.ds(base, RPW)], ididx_v, sem_st),
      pltpu.make_async_copy(txttab_hbm.at[pl.ds(0, 1)], row0_v, sem_st),
  ]
  for cp in st_copies:
    cp.start()

  def _zero(r, _):
    z = jnp.zeros((16,), jnp.float32)
    acc_v[r, pl.ds(0, 16)] = z
    acc_v[r, pl.ds(16, 16)] = z
    return _
  lax.fori_loop(0, RPW, _zero, None)

  for cp in st_copies:
    cp.wait()

  tx_copies = []
  for j in range(SEQ):
    cp = pltpu.make_async_copy(txttab_hbm.at[toks_v.at[j]], acc_v, sem_tx)
    cp.start(add=True)
    tx_copies.append(cp)

  def _gidx(g, _):
    r16 = pl.multiple_of(g * 16, 16)
    iv = ididx_v[pl.ds(r16, 16)]
    for cc in range(EMB):
      gidx_v[cc, pl.ds(r16, 16)] = iv + (cc * ID_V)
    return _
  lax.fori_loop(0, NG, _gidx, None)

  id_copies = []
  for cc in range(EMB):
    cp = pltpu.make_async_copy(
        idtabT_hbm.at[gidx_v.at[cc]], idcols_v.at[cc], sem_id)
    cp.start()
    id_copies.append(cp)

  one = jnp.ones((16,), jnp.float32)
  zero = jnp.zeros((16,), jnp.float32)

  def _count(g, _):
    r16 = pl.multiple_of(g * 16, 16)
    cnt = jnp.zeros((16,), jnp.float32)
    for j in range(SEQ):
      v = toks_v[j, pl.ds(r16, 16)]
      cnt = cnt + jnp.where(v != 0, one, zero)
    denom = jnp.maximum(cnt, 1.0)
    a_v[pl.ds(r16, 16)] = 1.0 / denom
    b_v[pl.ds(r16, 16)] = (float(SEQ) - cnt) / denom
    return _
  lax.fori_loop(0, NG, _count, None)

  for cp in tx_copies:
    cp.wait()

  r0_lo = row0_v[0, pl.ds(0, 16)]
  r0_hi = row0_v[0, pl.ds(16, 16)]
  lanes = lax.iota(jnp.int32, 16)

  def _scale(g, _):
    r16 = pl.multiple_of(g * 16, 16)
    rows = r16 + lanes
    avec = a_v[pl.ds(r16, 16)]
    bvec = b_v[pl.ds(r16, 16)]
    for cc in range(EMB):
      col = jnp.full((16,), cc, jnp.int32)
      accv = plsc.load_gather(acc_v, [rows, col])
      r0c = r0_lo[cc] if cc < 16 else r0_hi[cc - 16]
      txtT_v[cc, pl.ds(r16, 16)] = accv * avec - r0c * bvec
    return _
  lax.fori_loop(0, NG, _scale, None)

  wr_txt = pltpu.make_async_copy(
      txtT_v, outT_hbm.at[pl.ds(EMB, EMB), pl.ds(base, RPW)], sem_st)
  wr_txt.start()

  for cp in id_copies:
    cp.wait()
  wr_id = pltpu.make_async_copy(
      idcols_v, outT_hbm.at[pl.ds(0, EMB), pl.ds(base, RPW)], sem_st)
  wr_id.start()

  wr_txt.wait()
  wr_id.wait()


@functools.partial(jax.jit, static_argnums=())
def _run(jnrl_id_idx, tokT, idtabT_flat, text_table):
  mesh = plsc.VectorSubcoreMesh(core_axis_name="c", subcore_axis_name="s")
  f = pl.kernel(
      _sc_body,
      out_type=jax.ShapeDtypeStruct((2 * EMB, B), jnp.float32),
      mesh=mesh,
      compiler_params=pltpu.CompilerParams(
          use_tc_tiling_on_sc=False, needs_layout_passes=False),
      scratch_types=[
          pltpu.VMEM((SEQ, RPW), jnp.int32),
          pltpu.VMEM((RPW,), jnp.int32),
          pltpu.VMEM((EMB, RPW), jnp.int32),
          pltpu.VMEM((EMB, RPW), jnp.float32),
          pltpu.VMEM((RPW, EMB), jnp.float32),
          pltpu.VMEM((EMB, RPW), jnp.float32),
          pltpu.VMEM((RPW,), jnp.float32),
          pltpu.VMEM((RPW,), jnp.float32),
          pltpu.VMEM((1, EMB), jnp.float32),
          pltpu.SemaphoreType.DMA,
          pltpu.SemaphoreType.DMA,
          pltpu.SemaphoreType.DMA,
      ],
  )
  outT = f(jnrl_id_idx, tokT, idtabT_flat, text_table)
  return jnp.transpose(outT)


def kernel(jnrl_id_idx, text_token_ids, id_table, text_table):
  tokT = jnp.transpose(text_token_ids)
  idtabT_flat = jnp.transpose(id_table).reshape(-1)
  return _run(jnrl_id_idx, tokT, idtabT_flat, text_table)
